# Initial kernel scaffold; baseline (speedup 1.0000x reference)
import jax
import jax.numpy as jnp
from jax.experimental import pallas as pl


def kernel(edge_index, feature, W, b):
    raise NotImplementedError("write your pallas kernel here")


# speedup vs baseline: 5.9500x; 5.9500x over previous
import functools

import jax
import jax.numpy as jnp
from jax import lax
from jax.experimental import pallas as pl
from jax.experimental.pallas import tpu as pltpu
from jax.experimental.pallas import tpu_sc as plsc

N_NODES = 10000
N_EDGES = 320000
D = 128
DH = D // 2
DEGW = 8

N_CORES = 2
N_SUBCORES = 16
EDGES_PER_SUBCORE = N_EDGES // N_SUBCORES
CHUNK = 125
CHUNKS_PER_SUBCORE = EDGES_PER_SUBCORE // CHUNK
N_PAD = 10240
ROWS_PER_SUBCORE = N_PAD // N_SUBCORES


def _sc_aggregate(src3, dst3, feat2, zfeat, zdeg, ones_blk):
    mesh = plsc.VectorSubcoreMesh(core_axis_name="c", subcore_axis_name="s")

    @functools.partial(
        pl.kernel,
        mesh=mesh,
        compiler_params=pltpu.CompilerParams(use_tc_tiling_on_sc=False),
        out_type=[
            jax.ShapeDtypeStruct((N_CORES, N_PAD, DH), jnp.float32),
            jax.ShapeDtypeStruct((N_PAD, DEGW), jnp.float32),
        ],
        scratch_types=[
            pltpu.VMEM((CHUNKS_PER_SUBCORE, CHUNK), jnp.int32),
            pltpu.VMEM((CHUNKS_PER_SUBCORE, CHUNK), jnp.int32),
            pltpu.VMEM((CHUNK, DH), jnp.float32),
            pltpu.VMEM((CHUNK, DEGW), jnp.float32),
            pltpu.VMEM_SHARED((N_PAD, DH), jnp.float32),
            pltpu.VMEM_SHARED((N_PAD, DEGW), jnp.float32),
            pltpu.SemaphoreType.DMA,
        ],
    )
    def body(src_hbm, dst_hbm, feat_hbm, zf_hbm, zd_hbm, ones_hbm,
             agg_hbm, deg_hbm,
             idx_s, idx_d, rows, ones_v, acc_sh, deg_sh, sem):
        c = lax.axis_index("c")
        s = lax.axis_index("s")

        rbase = s * ROWS_PER_SUBCORE
        pltpu.sync_copy(zf_hbm, acc_sh.at[pl.ds(rbase, ROWS_PER_SUBCORE)])
        pltpu.sync_copy(zd_hbm, deg_sh.at[pl.ds(rbase, ROWS_PER_SUBCORE)])
        pltpu.sync_copy(ones_hbm, ones_v)
        pltpu.sync_copy(src_hbm.at[s], idx_s)
        pltpu.sync_copy(dst_hbm.at[s], idx_d)
        plsc.subcore_barrier()

        @pl.loop(0, CHUNKS_PER_SUBCORE)
        def _(i):
            pltpu.async_copy(feat_hbm.at[c].at[idx_s.at[i]], rows, sem).wait()
            pltpu.sync_copy(rows, acc_sh.at[idx_d.at[i]], add=True)

            @pl.when(c == 0)
            def _():
                pltpu.sync_copy(ones_v, deg_sh.at[idx_d.at[i]], add=True)

        plsc.subcore_barrier()
        pltpu.sync_copy(acc_sh.at[pl.ds(rbase, ROWS_PER_SUBCORE)],
                        agg_hbm.at[c, pl.ds(rbase, ROWS_PER_SUBCORE)])

        @pl.when(c == 0)
        def _():
            pltpu.sync_copy(deg_sh.at[pl.ds(rbase, ROWS_PER_SUBCORE)],
                            deg_hbm.at[pl.ds(rbase, ROWS_PER_SUBCORE)])

    return body(src3, dst3, feat2, zfeat, zdeg, ones_blk)


ROW_BLK = 1024


def _tc_body(p_ref, d_ref, wt_ref, bt_ref, o_ref):
    o_ref[...] = (
        jnp.dot(p_ref[0], wt_ref[0], preferred_element_type=jnp.float32)
        + jnp.dot(p_ref[1], wt_ref[1], preferred_element_type=jnp.float32)
        + jnp.dot(d_ref[...], bt_ref[...], preferred_element_type=jnp.float32)
    )


def _tc_combine(agg, deg, wt2, bt):
    grid = (N_PAD // ROW_BLK,)
    return pl.pallas_call(
        _tc_body,
        grid=grid,
        in_specs=[
            pl.BlockSpec((N_CORES, ROW_BLK, DH), lambda i: (0, i, 0)),
            pl.BlockSpec((ROW_BLK, DEGW), lambda i: (i, 0)),
            pl.BlockSpec((N_CORES, DH, D), lambda i: (0, 0, 0)),
            pl.BlockSpec((DEGW, D), lambda i: (0, 0)),
        ],
        out_specs=pl.BlockSpec((ROW_BLK, D), lambda i: (i, 0)),
        out_shape=jax.ShapeDtypeStruct((N_PAD, D), jnp.float32),
    )(agg, deg, wt2, bt)


def kernel(edge_index, feature, W, b):
    src = edge_index[0].astype(jnp.int32).reshape(
        N_SUBCORES, CHUNKS_PER_SUBCORE, CHUNK)
    dst = edge_index[1].astype(jnp.int32).reshape(
        N_SUBCORES, CHUNKS_PER_SUBCORE, CHUNK)
    feat2 = jnp.stack([feature[:, :DH], feature[:, DH:]])
    zfeat = jnp.zeros((ROWS_PER_SUBCORE, DH), jnp.float32)
    zdeg = jnp.zeros((ROWS_PER_SUBCORE, DEGW), jnp.float32)
    ones_blk = jnp.ones((CHUNK, DEGW), jnp.float32)
    agg, deg = _sc_aggregate(src, dst, feat2, zfeat, zdeg, ones_blk)
    wt = W.T
    wt2 = jnp.stack([wt[:DH], wt[DH:]])
    bt = jnp.broadcast_to(b / DEGW, (DEGW, D))
    return _tc_combine(agg, deg, wt2, bt)[:N_NODES]


# speedup vs baseline: 9.6271x; 1.6180x over previous
import functools

import jax
import jax.numpy as jnp
from jax import lax
from jax.experimental import pallas as pl
from jax.experimental.pallas import tpu as pltpu
from jax.experimental.pallas import tpu_sc as plsc

N_NODES = 10000
N_EDGES = 320000
D = 128
DH = D // 2
DEGW = 8

N_CORES = 2
N_SUBCORES = 16
EDGES_PER_SUBCORE = N_EDGES // N_SUBCORES
CHUNK = 125
CHUNKS_PER_SUBCORE = EDGES_PER_SUBCORE // CHUNK
HALF_CHUNKS = CHUNKS_PER_SUBCORE // 2
N_PAD = 10240
ROWS_PER_SUBCORE = N_PAD // N_SUBCORES


def _sc_aggregate(srcx, dst3, feat_rows, zfeat, zdeg, ones_blk):
    mesh = plsc.VectorSubcoreMesh(core_axis_name="c", subcore_axis_name="s")

    @functools.partial(
        pl.kernel,
        mesh=mesh,
        compiler_params=pltpu.CompilerParams(use_tc_tiling_on_sc=False),
        out_type=[
            jax.ShapeDtypeStruct((N_CORES, N_PAD, DH), jnp.float32),
            jax.ShapeDtypeStruct((N_CORES, N_PAD, DEGW), jnp.float32),
        ],
        scratch_types=[
            pltpu.VMEM((CHUNKS_PER_SUBCORE, CHUNK), jnp.int32),
            pltpu.VMEM((CHUNKS_PER_SUBCORE, CHUNK), jnp.int32),
            pltpu.VMEM((CHUNK, DH), jnp.float32),
            pltpu.VMEM((CHUNK, DH), jnp.float32),
            pltpu.VMEM((CHUNK, DEGW), jnp.float32),
            pltpu.VMEM_SHARED((N_PAD, DH), jnp.float32),
            pltpu.VMEM_SHARED((N_PAD, DEGW), jnp.float32),
            pltpu.SemaphoreType.DMA,
            pltpu.SemaphoreType.DMA,
        ],
    )
    def body(src_hbm, dst_hbm, feat_hbm, zf_hbm, zd_hbm, ones_hbm,
             agg_hbm, deg_hbm,
             idx_s, idx_d, rows0, rows1, ones_v, acc_sh, deg_sh, sem0, sem1):
        c = lax.axis_index("c")
        s = lax.axis_index("s")

        rbase = s * ROWS_PER_SUBCORE
        pltpu.sync_copy(zf_hbm, acc_sh.at[pl.ds(rbase, ROWS_PER_SUBCORE)])
        pltpu.sync_copy(zd_hbm, deg_sh.at[pl.ds(rbase, ROWS_PER_SUBCORE)])
        pltpu.sync_copy(ones_hbm, ones_v)
        pltpu.sync_copy(src_hbm.at[c, s], idx_s)
        pltpu.sync_copy(dst_hbm.at[s], idx_d)
        plsc.subcore_barrier()

        def fire(i, buf, sem):
            pltpu.async_copy(feat_hbm.at[idx_s.at[i]], buf, sem)

        def drain(buf, sem):
            pltpu.make_async_copy(feat_hbm.at[idx_s.at[0]], buf, sem).wait()

        def consume(i, buf):
            pltpu.sync_copy(buf, acc_sh.at[idx_d.at[i]], add=True)

            @pl.when((i // HALF_CHUNKS) == c)
            def _():
                pltpu.sync_copy(ones_v, deg_sh.at[idx_d.at[i]], add=True)

        fire(0, rows0, sem0)

        @pl.loop(0, CHUNKS_PER_SUBCORE - 2, step=2)
        def _(i):
            fire(i + 1, rows1, sem1)
            drain(rows0, sem0)
            consume(i, rows0)
            fire(i + 2, rows0, sem0)
            drain(rows1, sem1)
            consume(i + 1, rows1)

        fire(CHUNKS_PER_SUBCORE - 1, rows1, sem1)
        drain(rows0, sem0)
        consume(CHUNKS_PER_SUBCORE - 2, rows0)
        drain(rows1, sem1)
        consume(CHUNKS_PER_SUBCORE - 1, rows1)

        plsc.subcore_barrier()
        pltpu.sync_copy(acc_sh.at[pl.ds(rbase, ROWS_PER_SUBCORE)],
                        agg_hbm.at[c, pl.ds(rbase, ROWS_PER_SUBCORE)])
        pltpu.sync_copy(deg_sh.at[pl.ds(rbase, ROWS_PER_SUBCORE)],
                        deg_hbm.at[c, pl.ds(rbase, ROWS_PER_SUBCORE)])

    return body(srcx, dst3, feat_rows, zfeat, zdeg, ones_blk)


ROW_BLK = 1000


def _tc_body(p_ref, d_ref, wt_ref, bt_ref, o_ref):
    o_ref[...] = (
        jnp.dot(p_ref[0], wt_ref[0], preferred_element_type=jnp.float32)
        + jnp.dot(p_ref[1], wt_ref[1], preferred_element_type=jnp.float32)
        + jnp.dot(d_ref[0] + d_ref[1], bt_ref[...],
                  preferred_element_type=jnp.float32)
    )


def _tc_combine(agg, deg, wt2, bt):
    grid = (N_NODES // ROW_BLK,)
    return pl.pallas_call(
        _tc_body,
        grid=grid,
        in_specs=[
            pl.BlockSpec((N_CORES, ROW_BLK, DH), lambda i: (0, i, 0)),
            pl.BlockSpec((N_CORES, ROW_BLK, DEGW), lambda i: (0, i, 0)),
            pl.BlockSpec((N_CORES, DH, D), lambda i: (0, 0, 0)),
            pl.BlockSpec((DEGW, D), lambda i: (0, 0)),
        ],
        out_specs=pl.BlockSpec((ROW_BLK, D), lambda i: (i, 0)),
        out_shape=jax.ShapeDtypeStruct((N_NODES, D), jnp.float32),
    )(agg, deg, wt2, bt)


def kernel(edge_index, feature, W, b):
    src = edge_index[0].astype(jnp.int32).reshape(
        N_SUBCORES, CHUNKS_PER_SUBCORE, CHUNK)
    dst = edge_index[1].astype(jnp.int32).reshape(
        N_SUBCORES, CHUNKS_PER_SUBCORE, CHUNK)
    srcx = jnp.stack([2 * src, 2 * src + 1])
    feat_rows = feature.reshape(2 * N_NODES, DH)
    zfeat = jnp.zeros((ROWS_PER_SUBCORE, DH), jnp.float32)
    zdeg = jnp.zeros((ROWS_PER_SUBCORE, DEGW), jnp.float32)
    ones_blk = jnp.ones((CHUNK, DEGW), jnp.float32)
    agg, deg = _sc_aggregate(srcx, dst, feat_rows, zfeat, zdeg, ones_blk)
    wt = W.T
    wt2 = jnp.stack([wt[:DH], wt[DH:]])
    bt = jnp.broadcast_to(b / DEGW, (DEGW, D))
    return _tc_combine(agg, deg, wt2, bt)


# speedup vs baseline: 11.3405x; 1.1780x over previous
import functools

import jax
import jax.numpy as jnp
from jax import lax
from jax.experimental import pallas as pl
from jax.experimental.pallas import tpu as pltpu
from jax.experimental.pallas import tpu_sc as plsc

N_NODES = 10000
N_EDGES = 320000
D = 128
DH = D // 2
DEGW = 8

N_CORES = 2
N_SUBCORES = 16
EDGES_PER_SUBCORE = N_EDGES // N_SUBCORES
CHUNK = 125
CHUNKS_PER_SUBCORE = EDGES_PER_SUBCORE // CHUNK
HALF_CHUNKS = CHUNKS_PER_SUBCORE // 2
N_PAD = 10240
ROWS_PER_SUBCORE = N_PAD // N_SUBCORES


def _sc_aggregate(srcx, dst3, feat_rows, zfeat, zdeg, ones_blk):
    mesh = plsc.VectorSubcoreMesh(core_axis_name="c", subcore_axis_name="s")

    @functools.partial(
        pl.kernel,
        mesh=mesh,
        compiler_params=pltpu.CompilerParams(use_tc_tiling_on_sc=False),
        out_type=[
            jax.ShapeDtypeStruct((N_CORES, N_PAD, DH), jnp.float32),
            jax.ShapeDtypeStruct((N_CORES, N_PAD, DEGW), jnp.float32),
        ],
        scratch_types=[
            pltpu.VMEM((CHUNKS_PER_SUBCORE, CHUNK), jnp.int32),
            pltpu.VMEM((CHUNKS_PER_SUBCORE, CHUNK), jnp.int32),
            pltpu.VMEM((CHUNK, DH), jnp.float32),
            pltpu.VMEM((CHUNK, DH), jnp.float32),
            pltpu.VMEM((CHUNK, DH), jnp.float32),
            pltpu.VMEM((CHUNK, DH), jnp.float32),
            pltpu.VMEM((CHUNK, DEGW), jnp.float32),
            pltpu.VMEM_SHARED((N_PAD, DH), jnp.float32),
            pltpu.VMEM_SHARED((N_PAD, DEGW), jnp.float32),
            pltpu.SemaphoreType.DMA,
            pltpu.SemaphoreType.DMA,
            pltpu.SemaphoreType.DMA,
            pltpu.SemaphoreType.DMA,
        ],
    )
    def body(src_hbm, dst_hbm, feat_hbm, zf_hbm, zd_hbm, ones_hbm,
             agg_hbm, deg_hbm,
             idx_s, idx_d, rows0, rows1, rows2, rows3, ones_v,
             acc_sh, deg_sh, sem0, sem1, sem2, sem3):
        c = lax.axis_index("c")
        s = lax.axis_index("s")

        rbase = s * ROWS_PER_SUBCORE
        pltpu.sync_copy(zf_hbm, acc_sh.at[pl.ds(rbase, ROWS_PER_SUBCORE)])
        pltpu.sync_copy(zd_hbm, deg_sh.at[pl.ds(rbase, ROWS_PER_SUBCORE)])
        pltpu.sync_copy(ones_hbm, ones_v)
        pltpu.sync_copy(src_hbm.at[c, s], idx_s)
        pltpu.sync_copy(dst_hbm.at[s], idx_d)
        plsc.subcore_barrier()

        def fire(i, buf, sem):
            pltpu.async_copy(feat_hbm.at[idx_s.at[i]], buf, sem)

        def drain(buf, sem):
            pltpu.make_async_copy(feat_hbm.at[idx_s.at[0]], buf, sem).wait()

        def consume(i, buf):
            pltpu.sync_copy(buf, acc_sh.at[idx_d.at[i]], add=True)

            @pl.when((i // HALF_CHUNKS) == c)
            def _():
                pltpu.sync_copy(ones_v, deg_sh.at[idx_d.at[i]], add=True)

        bufs = (rows0, rows1, rows2, rows3)
        sems = (sem0, sem1, sem2, sem3)
        for j in range(3):
            fire(j, bufs[j], sems[j])

        @pl.loop(0, CHUNKS_PER_SUBCORE, step=4)
        def _(i):
            for j in range(4):
                drain(bufs[j], sems[j])
                consume(i + j, bufs[j])

                @pl.when(i + j + 3 < CHUNKS_PER_SUBCORE)
                def _():
                    fire(i + j + 3, bufs[(j + 3) % 4], sems[(j + 3) % 4])

        plsc.subcore_barrier()
        pltpu.sync_copy(acc_sh.at[pl.ds(rbase, ROWS_PER_SUBCORE)],
                        agg_hbm.at[c, pl.ds(rbase, ROWS_PER_SUBCORE)])
        pltpu.sync_copy(deg_sh.at[pl.ds(rbase, ROWS_PER_SUBCORE)],
                        deg_hbm.at[c, pl.ds(rbase, ROWS_PER_SUBCORE)])

    return body(srcx, dst3, feat_rows, zfeat, zdeg, ones_blk)


ROW_BLK = 1000


def _tc_body(p_ref, d_ref, wt_ref, bt_ref, o_ref):
    o_ref[...] = (
        jnp.dot(p_ref[0], wt_ref[0], preferred_element_type=jnp.float32)
        + jnp.dot(p_ref[1], wt_ref[1], preferred_element_type=jnp.float32)
        + jnp.dot(d_ref[0] + d_ref[1], bt_ref[...],
                  preferred_element_type=jnp.float32)
    )


def _tc_combine(agg, deg, wt2, bt):
    grid = (N_NODES // ROW_BLK,)
    return pl.pallas_call(
        _tc_body,
        grid=grid,
        in_specs=[
            pl.BlockSpec((N_CORES, ROW_BLK, DH), lambda i: (0, i, 0)),
            pl.BlockSpec((N_CORES, ROW_BLK, DEGW), lambda i: (0, i, 0)),
            pl.BlockSpec((N_CORES, DH, D), lambda i: (0, 0, 0)),
            pl.BlockSpec((DEGW, D), lambda i: (0, 0)),
        ],
        out_specs=pl.BlockSpec((ROW_BLK, D), lambda i: (i, 0)),
        out_shape=jax.ShapeDtypeStruct((N_NODES, D), jnp.float32),
    )(agg, deg, wt2, bt)


def kernel(edge_index, feature, W, b):
    src = edge_index[0].astype(jnp.int32).reshape(
        N_SUBCORES, CHUNKS_PER_SUBCORE, CHUNK)
    dst = edge_index[1].astype(jnp.int32).reshape(
        N_SUBCORES, CHUNKS_PER_SUBCORE, CHUNK)
    srcx = jnp.stack([2 * src, 2 * src + 1])
    feat_rows = feature.reshape(2 * N_NODES, DH)
    zfeat = jnp.zeros((ROWS_PER_SUBCORE, DH), jnp.float32)
    zdeg = jnp.zeros((ROWS_PER_SUBCORE, DEGW), jnp.float32)
    ones_blk = jnp.ones((CHUNK, DEGW), jnp.float32)
    agg, deg = _sc_aggregate(srcx, dst, feat_rows, zfeat, zdeg, ones_blk)
    wt = W.T
    wt2 = jnp.stack([wt[:DH], wt[DH:]])
    bt = jnp.broadcast_to(b / DEGW, (DEGW, D))
    return _tc_combine(agg, deg, wt2, bt)


# speedup vs baseline: 13.5448x; 1.1944x over previous
import functools

import jax
import jax.numpy as jnp
from jax import lax
from jax.experimental import pallas as pl
from jax.experimental.pallas import tpu as pltpu
from jax.experimental.pallas import tpu_sc as plsc

N_NODES = 10000
N_EDGES = 320000
D = 128
DH = D // 2
DEGW = 8

N_CORES = 2
N_SUBCORES = 16
EDGES_PER_SUBCORE = N_EDGES // N_SUBCORES
CHUNK = 125
CHUNKS_PER_SUBCORE = EDGES_PER_SUBCORE // CHUNK
HALF_CHUNKS = CHUNKS_PER_SUBCORE // 2
N_PAD = 10240
ROWS_PER_SUBCORE = N_PAD // N_SUBCORES


def _sc_aggregate(ei, feat_rows, zfeat, zdeg, ones_blk):
    mesh = plsc.VectorSubcoreMesh(core_axis_name="c", subcore_axis_name="s")

    @functools.partial(
        pl.kernel,
        mesh=mesh,
        compiler_params=pltpu.CompilerParams(use_tc_tiling_on_sc=False),
        out_type=[
            jax.ShapeDtypeStruct((N_CORES, N_PAD, DH), jnp.float32),
            jax.ShapeDtypeStruct((N_CORES, N_PAD, DEGW), jnp.float32),
        ],
        scratch_types=[
            pltpu.VMEM((CHUNKS_PER_SUBCORE, CHUNK), jnp.int32),
            pltpu.VMEM((CHUNKS_PER_SUBCORE, CHUNK), jnp.int32),
            pltpu.VMEM((CHUNK, DH), jnp.float32),
            pltpu.VMEM((CHUNK, DH), jnp.float32),
            pltpu.VMEM((CHUNK, DH), jnp.float32),
            pltpu.VMEM((CHUNK, DH), jnp.float32),
            pltpu.VMEM((CHUNK, DEGW), jnp.float32),
            pltpu.VMEM_SHARED((N_PAD, DH), jnp.float32),
            pltpu.VMEM_SHARED((N_PAD, DEGW), jnp.float32),
            pltpu.SemaphoreType.DMA,
            pltpu.SemaphoreType.DMA,
            pltpu.SemaphoreType.DMA,
            pltpu.SemaphoreType.DMA,
            pltpu.SemaphoreType.DMA,
            pltpu.SemaphoreType.DMA,
            pltpu.SemaphoreType.DMA,
            pltpu.SemaphoreType.DMA,
        ],
    )
    def body(ei_hbm, feat_hbm, zf_hbm, zd_hbm, ones_hbm,
             agg_hbm, deg_hbm,
             idx_s, idx_d, rows0, rows1, rows2, rows3, ones_v,
             acc_sh, deg_sh, sem0, sem1, sem2, sem3,
             ssem0, ssem1, ssem2, ssem3):
        c = lax.axis_index("c")
        s = lax.axis_index("s")

        rbase = s * ROWS_PER_SUBCORE
        pltpu.sync_copy(zf_hbm, acc_sh.at[pl.ds(rbase, ROWS_PER_SUBCORE)])
        pltpu.sync_copy(zd_hbm, deg_sh.at[pl.ds(rbase, ROWS_PER_SUBCORE)])
        pltpu.sync_copy(ones_hbm, ones_v)
        cbase = s * CHUNKS_PER_SUBCORE
        pltpu.sync_copy(ei_hbm.at[0, pl.ds(cbase, CHUNKS_PER_SUBCORE)], idx_s)
        pltpu.sync_copy(ei_hbm.at[1, pl.ds(cbase, CHUNKS_PER_SUBCORE)], idx_d)

        lane = lax.iota(jnp.int32, 16)

        @pl.loop(0, CHUNKS_PER_SUBCORE)
        def _(r):
            for k in range(7):
                sl = pl.ds(k * 16, 16)
                v = idx_s[r, sl]
                idx_s[r, sl] = v + v + c
            tl = pl.ds(109, 16)
            v = idx_s[r, tl]
            idx_s[r, tl] = jnp.where(lane < 3, v, v + v + c)

        plsc.subcore_barrier()

        def fire(i, buf, sem):
            pltpu.async_copy(feat_hbm.at[idx_s.at[i]], buf, sem)

        def drain(buf, sem):
            pltpu.make_async_copy(feat_hbm.at[idx_s.at[0]], buf, sem).wait()

        def consume(i, buf, ssem):
            pltpu.async_copy(buf, acc_sh.at[idx_d.at[i]], ssem, add=True)

            @pl.when((i // HALF_CHUNKS) == c)
            def _():
                pltpu.sync_copy(ones_v, deg_sh.at[idx_d.at[i]], add=True)

        def sdrain(buf, ssem):
            pltpu.make_async_copy(buf, acc_sh.at[idx_d.at[0]], ssem).wait()

        bufs = (rows0, rows1, rows2, rows3)
        sems = (sem0, sem1, sem2, sem3)
        ssems = (ssem0, ssem1, ssem2, ssem3)
        for j in range(3):
            fire(j, bufs[j], sems[j])

        @pl.loop(0, CHUNKS_PER_SUBCORE, step=4)
        def _(i):
            for j in range(4):
                drain(bufs[j], sems[j])
                consume(i + j, bufs[j], ssems[j])
                jj = (j + 3) % 4

                @pl.when(i + j + 3 < CHUNKS_PER_SUBCORE)
                def _():
                    @pl.when(i + j >= 1)
                    def _():
                        sdrain(bufs[jj], ssems[jj])

                    fire(i + j + 3, bufs[jj], sems[jj])

        for j in range(4):
            sdrain(bufs[j], ssems[j])

        plsc.subcore_barrier()
        pltpu.sync_copy(acc_sh.at[pl.ds(rbase, ROWS_PER_SUBCORE)],
                        agg_hbm.at[c, pl.ds(rbase, ROWS_PER_SUBCORE)])
        pltpu.sync_copy(deg_sh.at[pl.ds(rbase, ROWS_PER_SUBCORE)],
                        deg_hbm.at[c, pl.ds(rbase, ROWS_PER_SUBCORE)])

    return body(ei, feat_rows, zfeat, zdeg, ones_blk)


N_PAIR = N_NODES // 2
PAIR_BLK = 1000


def _tc_body(p_ref, d_ref, m_ref, c_ref, o_ref):
    deg_even = d_ref[0, :, 0:1] + d_ref[1, :, 0:1]
    deg_odd = d_ref[0, :, DEGW:DEGW + 1] + d_ref[1, :, DEGW:DEGW + 1]
    res = (
        jnp.dot(p_ref[0], m_ref[0], preferred_element_type=jnp.float32)
        + jnp.dot(p_ref[1], m_ref[1], preferred_element_type=jnp.float32)
        + deg_even * c_ref[0:1, :]
        + deg_odd * c_ref[1:2, :]
    )
    o_ref[...] = res.reshape(2 * PAIR_BLK, D)


def _tc_combine(aggp, degp, mm, cc):
    grid = (N_PAIR // PAIR_BLK,)
    return pl.pallas_call(
        _tc_body,
        grid=grid,
        in_specs=[
            pl.BlockSpec((N_CORES, PAIR_BLK, D), lambda i: (0, i, 0)),
            pl.BlockSpec((N_CORES, PAIR_BLK, 2 * DEGW), lambda i: (0, i, 0)),
            pl.BlockSpec((N_CORES, D, 2 * D), lambda i: (0, 0, 0)),
            pl.BlockSpec((N_CORES, 2 * D), lambda i: (0, 0)),
        ],
        out_specs=pl.BlockSpec((2 * PAIR_BLK, D), lambda i: (i, 0)),
        out_shape=jax.ShapeDtypeStruct((N_NODES, D), jnp.float32),
    )(aggp, degp, mm, cc)


def kernel(edge_index, feature, W, b):
    ei = edge_index.astype(jnp.int32).reshape(
        2, N_SUBCORES * CHUNKS_PER_SUBCORE, CHUNK)
    feat_rows = feature.reshape(2 * N_NODES, DH)
    zfeat = jnp.zeros((ROWS_PER_SUBCORE, DH), jnp.float32)
    zdeg = jnp.zeros((ROWS_PER_SUBCORE, DEGW), jnp.float32)
    ones_blk = jnp.ones((CHUNK, DEGW), jnp.float32)
    agg, deg = _sc_aggregate(ei, feat_rows, zfeat, zdeg, ones_blk)
    aggp = agg.reshape(N_CORES, N_PAD // 2, D)
    degp = deg.reshape(N_CORES, N_PAD // 2, 2 * DEGW)
    wt = W.T
    zblk = jnp.zeros((DH, D), jnp.float32)
    m0 = jnp.concatenate(
        [jnp.concatenate([wt[:DH], zblk], axis=1),
         jnp.concatenate([zblk, wt[:DH]], axis=1)], axis=0)
    m1 = jnp.concatenate(
        [jnp.concatenate([wt[DH:], zblk], axis=1),
         jnp.concatenate([zblk, wt[DH:]], axis=1)], axis=0)
    mm = jnp.stack([m0, m1])
    zb = jnp.zeros((D,), jnp.float32)
    cc = jnp.stack([jnp.concatenate([b, zb]), jnp.concatenate([zb, b])])
    return _tc_combine(aggp, degp, mm, cc)


# speedup vs baseline: 13.8731x; 1.0242x over previous
import functools

import jax
import jax.numpy as jnp
from jax import lax
from jax.experimental import pallas as pl
from jax.experimental.pallas import tpu as pltpu
from jax.experimental.pallas import tpu_sc as plsc

N_NODES = 10000
N_EDGES = 320000
D = 128
DH = D // 2
DEGW = 8

N_CORES = 2
N_SUBCORES = 16
EDGES_PER_SUBCORE = N_EDGES // N_SUBCORES
CHUNK = 125
CHUNKS_PER_SUBCORE = EDGES_PER_SUBCORE // CHUNK
HALF_CHUNKS = CHUNKS_PER_SUBCORE // 2
N_PAD = 10240
ROWS_PER_SUBCORE = N_PAD // N_SUBCORES


def _sc_aggregate(ei, feat_rows, zfeat, zdeg, ones_blk):
    mesh = plsc.VectorSubcoreMesh(core_axis_name="c", subcore_axis_name="s")

    @functools.partial(
        pl.kernel,
        mesh=mesh,
        compiler_params=pltpu.CompilerParams(use_tc_tiling_on_sc=False),
        out_type=[
            jax.ShapeDtypeStruct((N_CORES, N_PAD, DH), jnp.float32),
            jax.ShapeDtypeStruct((N_CORES, N_PAD, DEGW), jnp.float32),
        ],
        scratch_types=[
            pltpu.VMEM((CHUNKS_PER_SUBCORE, CHUNK), jnp.int32),
            pltpu.VMEM((CHUNKS_PER_SUBCORE, CHUNK), jnp.int32),
            pltpu.VMEM((CHUNK, DH), jnp.float32),
            pltpu.VMEM((CHUNK, DH), jnp.float32),
            pltpu.VMEM((CHUNK, DH), jnp.float32),
            pltpu.VMEM((CHUNK, DH), jnp.float32),
            pltpu.VMEM((CHUNK, DEGW), jnp.float32),
            pltpu.VMEM_SHARED((N_PAD, DH), jnp.float32),
            pltpu.VMEM_SHARED((N_PAD, DEGW), jnp.float32),
            pltpu.SemaphoreType.DMA,
            pltpu.SemaphoreType.DMA,
            pltpu.SemaphoreType.DMA,
            pltpu.SemaphoreType.DMA,
        ],
    )
    def body(ei_hbm, feat_hbm, zf_hbm, zd_hbm, ones_hbm,
             agg_hbm, deg_hbm,
             idx_s, idx_d, rows0, rows1, rows2, rows3, ones_v,
             acc_sh, deg_sh, sem0, sem1, sem2, sem3):
        c = lax.axis_index("c")
        s = lax.axis_index("s")

        rbase = s * ROWS_PER_SUBCORE
        pltpu.sync_copy(zf_hbm, acc_sh.at[pl.ds(rbase, ROWS_PER_SUBCORE)])
        pltpu.sync_copy(zd_hbm, deg_sh.at[pl.ds(rbase, ROWS_PER_SUBCORE)])
        pltpu.sync_copy(ones_hbm, ones_v)
        cbase = s * CHUNKS_PER_SUBCORE
        pltpu.sync_copy(ei_hbm.at[0, pl.ds(cbase, CHUNKS_PER_SUBCORE)], idx_s)
        pltpu.sync_copy(ei_hbm.at[1, pl.ds(cbase, CHUNKS_PER_SUBCORE)], idx_d)

        lane = lax.iota(jnp.int32, 16)

        @pl.loop(0, CHUNKS_PER_SUBCORE)
        def _(r):
            for k in range(7):
                sl = pl.ds(k * 16, 16)
                v = idx_s[r, sl]
                idx_s[r, sl] = v + v + c
            tl = pl.ds(109, 16)
            v = idx_s[r, tl]
            idx_s[r, tl] = jnp.where(lane < 3, v, v + v + c)

        plsc.subcore_barrier()

        def fire(i, buf, sem):
            pltpu.async_copy(feat_hbm.at[idx_s.at[i]], buf, sem)

        def drain(buf, sem):
            pltpu.make_async_copy(feat_hbm.at[idx_s.at[0]], buf, sem).wait()

        def consume(i, buf):
            pltpu.sync_copy(buf, acc_sh.at[idx_d.at[i]], add=True)

            @pl.when((i // HALF_CHUNKS) == c)
            def _():
                pltpu.sync_copy(ones_v, deg_sh.at[idx_d.at[i]], add=True)

        bufs = (rows0, rows1, rows2, rows3)
        sems = (sem0, sem1, sem2, sem3)
        for j in range(3):
            fire(j, bufs[j], sems[j])

        @pl.loop(0, CHUNKS_PER_SUBCORE, step=4)
        def _(i):
            for j in range(4):
                drain(bufs[j], sems[j])
                consume(i + j, bufs[j])

                @pl.when(i + j + 3 < CHUNKS_PER_SUBCORE)
                def _():
                    fire(i + j + 3, bufs[(j + 3) % 4], sems[(j + 3) % 4])

        plsc.subcore_barrier()
        pltpu.sync_copy(acc_sh.at[pl.ds(rbase, ROWS_PER_SUBCORE)],
                        agg_hbm.at[c, pl.ds(rbase, ROWS_PER_SUBCORE)])
        pltpu.sync_copy(deg_sh.at[pl.ds(rbase, ROWS_PER_SUBCORE)],
                        deg_hbm.at[c, pl.ds(rbase, ROWS_PER_SUBCORE)])

    return body(ei, feat_rows, zfeat, zdeg, ones_blk)


N_PAIR = N_NODES // 2
PAIR_BLK = 1000


def _tc_body(p_ref, d_ref, m_ref, c_ref, o_ref):
    deg_even = d_ref[0, :, 0:1] + d_ref[1, :, 0:1]
    deg_odd = d_ref[0, :, DEGW:DEGW + 1] + d_ref[1, :, DEGW:DEGW + 1]
    res = (
        jnp.dot(p_ref[0], m_ref[0], preferred_element_type=jnp.float32)
        + jnp.dot(p_ref[1], m_ref[1], preferred_element_type=jnp.float32)
        + deg_even * c_ref[0:1, :]
        + deg_odd * c_ref[1:2, :]
    )
    o_ref[...] = res.reshape(2 * PAIR_BLK, D)


def _tc_combine(aggp, degp, mm, cc):
    grid = (N_PAIR // PAIR_BLK,)
    return pl.pallas_call(
        _tc_body,
        grid=grid,
        in_specs=[
            pl.BlockSpec((N_CORES, PAIR_BLK, D), lambda i: (0, i, 0)),
            pl.BlockSpec((N_CORES, PAIR_BLK, 2 * DEGW), lambda i: (0, i, 0)),
            pl.BlockSpec((N_CORES, D, 2 * D), lambda i: (0, 0, 0)),
            pl.BlockSpec((N_CORES, 2 * D), lambda i: (0, 0)),
        ],
        out_specs=pl.BlockSpec((2 * PAIR_BLK, D), lambda i: (i, 0)),
        out_shape=jax.ShapeDtypeStruct((N_NODES, D), jnp.float32),
    )(aggp, degp, mm, cc)


def kernel(edge_index, feature, W, b):
    ei = edge_index.astype(jnp.int32).reshape(
        2, N_SUBCORES * CHUNKS_PER_SUBCORE, CHUNK)
    feat_rows = feature.reshape(2 * N_NODES, DH)
    zfeat = jnp.zeros((ROWS_PER_SUBCORE, DH), jnp.float32)
    zdeg = jnp.zeros((ROWS_PER_SUBCORE, DEGW), jnp.float32)
    ones_blk = jnp.ones((CHUNK, DEGW), jnp.float32)
    agg, deg = _sc_aggregate(ei, feat_rows, zfeat, zdeg, ones_blk)
    aggp = agg.reshape(N_CORES, N_PAD // 2, D)
    degp = deg.reshape(N_CORES, N_PAD // 2, 2 * DEGW)
    wt = W.T
    zblk = jnp.zeros((DH, D), jnp.float32)
    m0 = jnp.concatenate(
        [jnp.concatenate([wt[:DH], zblk], axis=1),
         jnp.concatenate([zblk, wt[:DH]], axis=1)], axis=0)
    m1 = jnp.concatenate(
        [jnp.concatenate([wt[DH:], zblk], axis=1),
         jnp.concatenate([zblk, wt[DH:]], axis=1)], axis=0)
    mm = jnp.stack([m0, m1])
    zb = jnp.zeros((D,), jnp.float32)
    cc = jnp.stack([jnp.concatenate([b, zb]), jnp.concatenate([zb, b])])
    return _tc_combine(aggp, degp, mm, cc)
